# baseline (speedup 1.0000x reference)
import functools

import jax
import jax.numpy as jnp
from jax import lax
from jax.experimental import pallas as pl
from jax.experimental.pallas import tpu as pltpu
from jax.experimental.pallas import tpu_sc as plsc

N_NODES = 10000
HIDDEN = 128
OUT_DIM = 1
NUM_GRAPHS = 256

NC = 2
NS = 16
NW = NC * NS
RPW = 320
NCHUNK = 3
DUMMY = NUM_GRAPHS
ACC_ROWS = 384
ZROWS = ACC_ROWS // NS
LAST_W = NW - 1
LAST_ROWS = N_NODES - LAST_W * RPW


def _sc_segment_sum(h_v, batch, zeros_acc):
    mesh = plsc.VectorSubcoreMesh(core_axis_name="c", subcore_axis_name="s",
                                  num_cores=NC, num_subcores=NS)

    @functools.partial(
        pl.kernel,
        out_type=jax.ShapeDtypeStruct((NC, NUM_GRAPHS, HIDDEN), jnp.float32),
        mesh=mesh,
    scratch_types=[
            pltpu.VMEM((NCHUNK * 128, HIDDEN), jnp.float32),
            pltpu.VMEM((NCHUNK, 128), jnp.int32),
            pltpu.VMEM_SHARED((ACC_ROWS, HIDDEN), jnp.float32),
            pltpu.SemaphoreType.DMA,
            pltpu.SemaphoreType.DMA,
            pltpu.SemaphoreType.DMA,
            pltpu.SemaphoreType.DMA,
            pltpu.SemaphoreType.DMA,
        ],
    )
    def seg_sum(h_hbm, b_hbm, z_hbm, sums_out,
                hbuf, idxbuf, acc,
                s0, s1, s2, isem, ssem):
        c = lax.axis_index("c")
        s = lax.axis_index("s")
        w = c * NS + s
        base = w * RPW
        hsems = [s0, s1, s2]
        dummy16 = jnp.full((16,), DUMMY, jnp.int32)
        LB = N_NODES - 128
        PRE = RPW * LAST_W - LB

        @pl.when(w < LAST_W)
        def _():
            icp = [pltpu.async_copy(b_hbm.at[pl.ds(base + j * 128, 128)],
                                    idxbuf.at[j], isem)
                   for j in range(NCHUNK)]
            pltpu.sync_copy(z_hbm, acc.at[pl.ds(s * ZROWS, ZROWS)])
            for cp in icp:
                cp.wait()
            for k in range((RPW - 256) // 16, 128 // 16):
                idxbuf[NCHUNK - 1, pl.ds(k * 16, 16)] = dummy16

        @pl.when(w == LAST_W)
        def _():
            icp = pltpu.async_copy(b_hbm.at[pl.ds(LB, 128)], idxbuf.at[0],
                                   isem)
            pltpu.sync_copy(z_hbm, acc.at[pl.ds(s * ZROWS, ZROWS)])
            icp.wait()
            for k in range(PRE // 16):
                idxbuf[0, pl.ds(k * 16, 16)] = dummy16

        plsc.subcore_barrier()

        @pl.when(w < LAST_W)
        def _():
            h_cps = [pltpu.async_copy(
                h_hbm.at[pl.ds(base + j * 128, 128)],
                hbuf.at[pl.ds(j * 128, 128)], hsems[j])
                for j in range(NCHUNK)]
            sc_cps = []
            for j in range(NCHUNK):
                h_cps[j].wait()
                sc_cps.append(pltpu.async_copy(
                    hbuf.at[pl.ds(j * 128, 128)],
                    acc.at[idxbuf.at[j]], ssem, add=True))
            for cp in sc_cps:
                cp.wait()

        @pl.when(w == LAST_W)
        def _():
            pltpu.async_copy(h_hbm.at[pl.ds(LB, 128)],
                             hbuf.at[pl.ds(0, 128)], s0).wait()
            pltpu.async_copy(hbuf.at[pl.ds(0, 128)],
                             acc.at[idxbuf.at[0]], ssem, add=True).wait()

        plsc.subcore_barrier()

        pltpu.sync_copy(acc.at[pl.ds(s * 16, 16)],
                        sums_out.at[c, pl.ds(s * 16, 16)])

    return seg_sum(h_v, batch, zeros_acc)


def _counts_kernel(b_ref, out_ref):
    seg = lax.broadcasted_iota(jnp.int32, (NUM_GRAPHS, 1), 0)
    onehot = (b_ref[...] == seg).astype(jnp.float32)
    out_ref[...] = jnp.sum(onehot, axis=1, keepdims=True)


def _mlp_kernel(s_ref, c_ref, w1_ref, b1_ref, w2_ref, b2_ref, w3_ref, b3_ref,
                out_ref):
    sums = s_ref[0] + s_ref[1]
    counts = c_ref[...]
    pooled = sums / jnp.maximum(counts, 1.0)
    x = jnp.maximum(
        jnp.dot(pooled, w1_ref[...], preferred_element_type=jnp.float32)
        + b1_ref[...], 0.0)
    x = jnp.maximum(
        jnp.dot(x, w2_ref[...], preferred_element_type=jnp.float32)
        + b2_ref[...], 0.0)
    pred = jnp.dot(x, w3_ref[...], preferred_element_type=jnp.float32) \
        + b3_ref[...]
    out_ref[...] = jnp.where(counts > 0.0, pred, 0.0)


def kernel(h_v, edge_index, batch, W1, b1, W2, b2, W3, b3):
    del edge_index
    b32 = batch.astype(jnp.int32)
    counts = pl.pallas_call(
        _counts_kernel,
        out_shape=jax.ShapeDtypeStruct((NUM_GRAPHS, 1), jnp.float32),
    )(b32.reshape(1, N_NODES))
    sums = _sc_segment_sum(h_v, b32, jnp.zeros((ZROWS, HIDDEN), jnp.float32))
    return pl.pallas_call(
        _mlp_kernel,
        out_shape=jax.ShapeDtypeStruct((NUM_GRAPHS, OUT_DIM), jnp.float32),
    )(sums, counts, W1, b1.reshape(1, HIDDEN), W2, b2.reshape(1, HIDDEN),
      W3, b3.reshape(1, OUT_DIM))


# speedup vs baseline: 1.0697x; 1.0697x over previous
import functools

import jax
import jax.numpy as jnp
import numpy as np
from jax import lax
from jax.experimental import pallas as pl
from jax.experimental.pallas import tpu as pltpu
from jax.experimental.pallas import tpu_sc as plsc

N_NODES = 10000
HIDDEN = 128
OUT_DIM = 1
NUM_GRAPHS = 256

NC = 2
NS = 16
NW = NC * NS
RPW = 320
NCHUNK = 3
DUMMY = NUM_GRAPHS
ACC_ROWS = 384
ZROWS = ACC_ROWS // NS
LAST_W = NW - 1
LAST_ROWS = N_NODES - LAST_W * RPW


def _sc_segment_sum(h_v, batch, zeros_acc):
    mesh = plsc.VectorSubcoreMesh(core_axis_name="c", subcore_axis_name="s",
                                  num_cores=NC, num_subcores=NS)

    @functools.partial(
        pl.kernel,
        out_type=jax.ShapeDtypeStruct((NC, NUM_GRAPHS, HIDDEN), jnp.float32),
        mesh=mesh,
    scratch_types=[
            pltpu.VMEM((NCHUNK * 128, HIDDEN), jnp.float32),
            pltpu.VMEM((NCHUNK, 128), jnp.int32),
            pltpu.VMEM_SHARED((ACC_ROWS, HIDDEN), jnp.float32),
            pltpu.SemaphoreType.DMA,
            pltpu.SemaphoreType.DMA,
            pltpu.SemaphoreType.DMA,
            pltpu.SemaphoreType.DMA,
            pltpu.SemaphoreType.DMA,
        ],
    )
    def seg_sum(h_hbm, b_hbm, z_hbm, sums_out,
                hbuf, idxbuf, acc,
                s0, s1, s2, isem, ssem):
        c = lax.axis_index("c")
        s = lax.axis_index("s")
        w = c * NS + s
        base = w * RPW
        hsems = [s0, s1, s2]
        dummy16 = jnp.full((16,), DUMMY, jnp.int32)
        LB = N_NODES - 128
        PRE = RPW * LAST_W - LB

        @pl.when(w < LAST_W)
        def _():
            icp = [pltpu.async_copy(b_hbm.at[pl.ds(base + j * 128, 128)],
                                    idxbuf.at[j], isem)
                   for j in range(NCHUNK)]
            pltpu.sync_copy(z_hbm.at[pl.ds(s * ZROWS, ZROWS)],
                            acc.at[pl.ds(s * ZROWS, ZROWS)])
            for cp in icp:
                cp.wait()
            for k in range((RPW - 256) // 16, 128 // 16):
                idxbuf[NCHUNK - 1, pl.ds(k * 16, 16)] = dummy16

        @pl.when(w == LAST_W)
        def _():
            icp = pltpu.async_copy(b_hbm.at[pl.ds(LB, 128)], idxbuf.at[0],
                                   isem)
            pltpu.sync_copy(z_hbm.at[pl.ds(s * ZROWS, ZROWS)],
                            acc.at[pl.ds(s * ZROWS, ZROWS)])
            icp.wait()
            for k in range(PRE // 16):
                idxbuf[0, pl.ds(k * 16, 16)] = dummy16

        plsc.subcore_barrier()

        @pl.when(w < LAST_W)
        def _():
            h_cps = [pltpu.async_copy(
                h_hbm.at[pl.ds(base + j * 128, 128)],
                hbuf.at[pl.ds(j * 128, 128)], hsems[j])
                for j in range(NCHUNK)]
            sc_cps = []
            for j in range(NCHUNK):
                h_cps[j].wait()
                sc_cps.append(pltpu.async_copy(
                    hbuf.at[pl.ds(j * 128, 128)],
                    acc.at[idxbuf.at[j]], ssem, add=True))
            for cp in sc_cps:
                cp.wait()

        @pl.when(w == LAST_W)
        def _():
            pltpu.async_copy(h_hbm.at[pl.ds(LB, 128)],
                             hbuf.at[pl.ds(0, 128)], s0).wait()
            pltpu.async_copy(hbuf.at[pl.ds(0, 128)],
                             acc.at[idxbuf.at[0]], ssem, add=True).wait()

        plsc.subcore_barrier()

        pltpu.sync_copy(acc.at[pl.ds(s * 16, 16)],
                        sums_out.at[c, pl.ds(s * 16, 16)])

    return seg_sum(h_v, batch, zeros_acc)


def _counts_kernel(b_ref, out_ref):
    seg = lax.broadcasted_iota(jnp.int32, (NUM_GRAPHS, 1), 0)
    onehot = (b_ref[...] == seg).astype(jnp.float32)
    out_ref[...] = jnp.sum(onehot, axis=1, keepdims=True)


def _mlp_kernel(s_ref, c_ref, w1_ref, b1_ref, w2_ref, b2_ref, w3_ref, b3_ref,
                out_ref):
    sums = s_ref[0] + s_ref[1]
    counts = c_ref[...]
    pooled = sums / jnp.maximum(counts, 1.0)
    x = jnp.maximum(
        jnp.dot(pooled, w1_ref[...], preferred_element_type=jnp.float32)
        + b1_ref[...], 0.0)
    x = jnp.maximum(
        jnp.dot(x, w2_ref[...], preferred_element_type=jnp.float32)
        + b2_ref[...], 0.0)
    pred = jnp.dot(x, w3_ref[...], preferred_element_type=jnp.float32) \
        + b3_ref[...]
    pred = jnp.where(counts > 0.0, pred, 0.0)
    r = lax.broadcasted_iota(jnp.int32, (NUM_GRAPHS, NUM_GRAPHS), 0)
    q = lax.broadcasted_iota(jnp.int32, (NUM_GRAPHS, NUM_GRAPHS), 1)
    eye = (r == q).astype(jnp.float32)
    out_ref[...] = lax.dot_general(
        pred, eye, dimension_numbers=(((0,), (0,)), ((), ())),
        preferred_element_type=jnp.float32)


def kernel(h_v, edge_index, batch, W1, b1, W2, b2, W3, b3):
    del edge_index
    b32 = batch.astype(jnp.int32)
    counts = pl.pallas_call(
        _counts_kernel,
        out_shape=jax.ShapeDtypeStruct((NUM_GRAPHS, 1), jnp.float32),
    )(b32.reshape(1, N_NODES))
    sums = _sc_segment_sum(h_v, b32,
                           jnp.asarray(np.zeros((ACC_ROWS, HIDDEN),
                                                np.float32)))
    pred_row = pl.pallas_call(
        _mlp_kernel,
        out_shape=jax.ShapeDtypeStruct((1, NUM_GRAPHS), jnp.float32),
    )(sums, counts, W1, b1.reshape(1, HIDDEN), W2, b2.reshape(1, HIDDEN),
      W3, b3.reshape(1, OUT_DIM))
    return pred_row.reshape(NUM_GRAPHS, OUT_DIM)


# speedup vs baseline: 1.0891x; 1.0182x over previous
import functools

import jax
import jax.numpy as jnp
import numpy as np
from jax import lax
from jax.experimental import pallas as pl
from jax.experimental.pallas import tpu as pltpu
from jax.experimental.pallas import tpu_sc as plsc

N_NODES = 10000
HIDDEN = 128
OUT_DIM = 1
NUM_GRAPHS = 256

NC = 2
NS = 16
NW = NC * NS
RPW = 320
NCHUNK = 3
DUMMY = NUM_GRAPHS
ACC_ROWS = 384
ZROWS = ACC_ROWS // NS
LAST_W = NW - 1
LAST_ROWS = N_NODES - LAST_W * RPW


def _zero_stripe(hbuf, acc, s):
    zero16 = jnp.zeros((16,), jnp.float32)

    def body(r, _):
        for j in range(HIDDEN // 16):
            hbuf[r, pl.ds(j * 16, 16)] = zero16
        return 0

    lax.fori_loop(0, ZROWS, body, 0)
    pltpu.sync_copy(hbuf.at[pl.ds(0, ZROWS)],
                    acc.at[pl.ds(s * ZROWS, ZROWS)])


def _sc_segment_sum(h_v, batch):
    mesh = plsc.VectorSubcoreMesh(core_axis_name="c", subcore_axis_name="s",
                                  num_cores=NC, num_subcores=NS)

    @functools.partial(
        pl.kernel,
        out_type=jax.ShapeDtypeStruct((NC, NUM_GRAPHS, HIDDEN), jnp.float32),
        mesh=mesh,
    scratch_types=[
            pltpu.VMEM((NCHUNK * 128, HIDDEN), jnp.float32),
            pltpu.VMEM((NCHUNK, 128), jnp.int32),
            pltpu.VMEM_SHARED((ACC_ROWS, HIDDEN), jnp.float32),
            pltpu.SemaphoreType.DMA,
            pltpu.SemaphoreType.DMA,
            pltpu.SemaphoreType.DMA,
            pltpu.SemaphoreType.DMA,
            pltpu.SemaphoreType.DMA,
        ],
    )
    def seg_sum(h_hbm, b_hbm, sums_out,
                hbuf, idxbuf, acc,
                s0, s1, s2, isem, ssem):
        c = lax.axis_index("c")
        s = lax.axis_index("s")
        w = c * NS + s
        base = w * RPW
        hsems = [s0, s1, s2]
        dummy16 = jnp.full((16,), DUMMY, jnp.int32)
        LB = N_NODES - 128
        PRE = RPW * LAST_W - LB

        @pl.when(w < LAST_W)
        def _():
            icp = [pltpu.async_copy(b_hbm.at[pl.ds(base + j * 128, 128)],
                                    idxbuf.at[j], isem)
                   for j in range(NCHUNK)]
            _zero_stripe(hbuf, acc, s)
            for cp in icp:
                cp.wait()
            for k in range((RPW - 256) // 16, 128 // 16):
                idxbuf[NCHUNK - 1, pl.ds(k * 16, 16)] = dummy16

        @pl.when(w == LAST_W)
        def _():
            icp = pltpu.async_copy(b_hbm.at[pl.ds(LB, 128)], idxbuf.at[0],
                                   isem)
            _zero_stripe(hbuf, acc, s)
            icp.wait()
            for k in range(PRE // 16):
                idxbuf[0, pl.ds(k * 16, 16)] = dummy16

        plsc.subcore_barrier()

        @pl.when(w < LAST_W)
        def _():
            h_cps = [pltpu.async_copy(
                h_hbm.at[pl.ds(base + j * 128, 128)],
                hbuf.at[pl.ds(j * 128, 128)], hsems[j])
                for j in range(NCHUNK)]
            sc_cps = []
            for j in range(NCHUNK):
                h_cps[j].wait()
                sc_cps.append(pltpu.async_copy(
                    hbuf.at[pl.ds(j * 128, 128)],
                    acc.at[idxbuf.at[j]], ssem, add=True))
            for cp in sc_cps:
                cp.wait()

        @pl.when(w == LAST_W)
        def _():
            pltpu.async_copy(h_hbm.at[pl.ds(LB, 128)],
                             hbuf.at[pl.ds(0, 128)], s0).wait()
            pltpu.async_copy(hbuf.at[pl.ds(0, 128)],
                             acc.at[idxbuf.at[0]], ssem, add=True).wait()

        plsc.subcore_barrier()

        pltpu.sync_copy(acc.at[pl.ds(s * 16, 16)],
                        sums_out.at[c, pl.ds(s * 16, 16)])

    return seg_sum(h_v, batch)


def _counts_kernel(b_ref, out_ref):
    seg = lax.broadcasted_iota(jnp.int32, (NUM_GRAPHS, 1), 0)
    onehot = (b_ref[...] == seg).astype(jnp.float32)
    out_ref[...] = jnp.sum(onehot, axis=1, keepdims=True)


def _mlp_kernel(s_ref, c_ref, w1_ref, b1_ref, w2_ref, b2_ref, w3_ref, b3_ref,
                out_ref):
    sums = s_ref[0] + s_ref[1]
    counts = c_ref[...]
    pooled = sums / jnp.maximum(counts, 1.0)
    x = jnp.maximum(
        jnp.dot(pooled, w1_ref[...], preferred_element_type=jnp.float32)
        + b1_ref[...], 0.0)
    x = jnp.maximum(
        jnp.dot(x, w2_ref[...], preferred_element_type=jnp.float32)
        + b2_ref[...], 0.0)
    pred = jnp.dot(x, w3_ref[...], preferred_element_type=jnp.float32) \
        + b3_ref[...]
    pred = jnp.where(counts > 0.0, pred, 0.0)
    r = lax.broadcasted_iota(jnp.int32, (NUM_GRAPHS, NUM_GRAPHS), 0)
    q = lax.broadcasted_iota(jnp.int32, (NUM_GRAPHS, NUM_GRAPHS), 1)
    eye = (r == q).astype(jnp.float32)
    out_ref[...] = lax.dot_general(
        pred, eye, dimension_numbers=(((0,), (0,)), ((), ())),
        preferred_element_type=jnp.float32)


def kernel(h_v, edge_index, batch, W1, b1, W2, b2, W3, b3):
    del edge_index
    b32 = batch.astype(jnp.int32)
    counts = pl.pallas_call(
        _counts_kernel,
        out_shape=jax.ShapeDtypeStruct((NUM_GRAPHS, 1), jnp.float32),
    )(b32.reshape(1, N_NODES))
    sums = _sc_segment_sum(h_v, b32)
    pred_row = pl.pallas_call(
        _mlp_kernel,
        out_shape=jax.ShapeDtypeStruct((1, NUM_GRAPHS), jnp.float32),
    )(sums, counts, W1, b1.reshape(1, HIDDEN), W2, b2.reshape(1, HIDDEN),
      W3, b3.reshape(1, OUT_DIM))
    return pred_row.reshape(NUM_GRAPHS, OUT_DIM)


# speedup vs baseline: 1.0937x; 1.0042x over previous
import functools

import jax
import jax.numpy as jnp
from jax import lax
from jax.experimental import pallas as pl
from jax.experimental.pallas import tpu as pltpu
from jax.experimental.pallas import tpu_sc as plsc

N_NODES = 10000
HIDDEN = 128
OUT_DIM = 1
NUM_GRAPHS = 256

NC = 2
NS = 16
NW = NC * NS
RPW = 320
NCHUNK = 3
DUMMY = NUM_GRAPHS
ACC_ROWS = 384
ZROWS = ACC_ROWS // NS
LAST_W = NW - 1


def _zero_stripe(hbuf, acc, s):
    zero16 = jnp.zeros((16,), jnp.float32)

    def body(r, _):
        for j in range(HIDDEN // 16):
            hbuf[r, pl.ds(j * 16, 16)] = zero16
        return 0

    lax.fori_loop(0, ZROWS, body, 0)
    pltpu.sync_copy(hbuf.at[pl.ds(0, ZROWS)],
                    acc.at[pl.ds(s * ZROWS, ZROWS)])


def _sc_segment_sum(h_v, batch):
    mesh = plsc.VectorSubcoreMesh(core_axis_name="c", subcore_axis_name="s",
                                  num_cores=NC, num_subcores=NS)

    @functools.partial(
        pl.kernel,
        out_type=jax.ShapeDtypeStruct((NC, NUM_GRAPHS, HIDDEN), jnp.float32),
        mesh=mesh,
    scratch_types=[
            pltpu.VMEM((NCHUNK * 128, HIDDEN), jnp.float32),
            pltpu.VMEM((NCHUNK, 128), jnp.int32),
            pltpu.VMEM_SHARED((ACC_ROWS, HIDDEN), jnp.float32),
            pltpu.SemaphoreType.DMA,
            pltpu.SemaphoreType.DMA,
            pltpu.SemaphoreType.DMA,
            pltpu.SemaphoreType.DMA,
            pltpu.SemaphoreType.DMA,
        ],
    )
    def seg_sum(h_hbm, b_hbm, sums_out,
                hbuf, idxbuf, acc,
                s0, s1, s2, isem, ssem):
        c = lax.axis_index("c")
        s = lax.axis_index("s")
        w = c * NS + s
        base = w * RPW
        hsems = [s0, s1, s2]
        dummy16 = jnp.full((16,), DUMMY, jnp.int32)
        LB = N_NODES - 128
        PRE = RPW * LAST_W - LB

        @pl.when(w < LAST_W)
        def _():
            icp = [pltpu.async_copy(b_hbm.at[pl.ds(base + j * 128, 128)],
                                    idxbuf.at[j], isem)
                   for j in range(NCHUNK)]
            _zero_stripe(hbuf, acc, s)
            for cp in icp:
                cp.wait()
            for k in range((RPW - 256) // 16, 128 // 16):
                idxbuf[NCHUNK - 1, pl.ds(k * 16, 16)] = dummy16

        @pl.when(w == LAST_W)
        def _():
            icp = pltpu.async_copy(b_hbm.at[pl.ds(LB, 128)], idxbuf.at[0],
                                   isem)
            _zero_stripe(hbuf, acc, s)
            icp.wait()
            for k in range(PRE // 16):
                idxbuf[0, pl.ds(k * 16, 16)] = dummy16

        plsc.subcore_barrier()

        @pl.when(w < LAST_W)
        def _():
            h_cps = [pltpu.async_copy(
                h_hbm.at[pl.ds(base + j * 128, 128)],
                hbuf.at[pl.ds(j * 128, 128)], hsems[j])
                for j in range(NCHUNK)]
            sc_cps = []
            for j in range(NCHUNK):
                h_cps[j].wait()
                sc_cps.append(pltpu.async_copy(
                    hbuf.at[pl.ds(j * 128, 128)],
                    acc.at[idxbuf.at[j]], ssem, add=True))
            for cp in sc_cps:
                cp.wait()

        @pl.when(w == LAST_W)
        def _():
            pltpu.async_copy(h_hbm.at[pl.ds(LB, 128)],
                             hbuf.at[pl.ds(0, 128)], s0).wait()
            pltpu.async_copy(hbuf.at[pl.ds(0, 128)],
                             acc.at[idxbuf.at[0]], ssem, add=True).wait()

        plsc.subcore_barrier()

        pltpu.sync_copy(acc.at[pl.ds(s * 16, 16)],
                        sums_out.at[c, pl.ds(s * 16, 16)])

    return seg_sum(h_v, batch)


def _counts_kernel(b_ref, out_ref):
    seg = lax.broadcasted_iota(jnp.int32, (NUM_GRAPHS, 1), 0)
    onehot = (b_ref[...] == seg).astype(jnp.float32)
    out_ref[...] = jnp.sum(onehot, axis=1, keepdims=True)


def _mlp_kernel(s_ref, c_ref, w1_ref, b1_ref, w2_ref, b2_ref, w3_ref, b3_ref,
                out_ref):
    sums = s_ref[0] + s_ref[1]
    counts = c_ref[...]
    pooled = sums / jnp.maximum(counts, 1.0)
    x = jnp.maximum(
        jnp.dot(pooled, w1_ref[...], preferred_element_type=jnp.float32)
        + b1_ref[...], 0.0)
    x = jnp.maximum(
        jnp.dot(x, w2_ref[...], preferred_element_type=jnp.float32)
        + b2_ref[...], 0.0)
    pred = jnp.dot(x, w3_ref[...], preferred_element_type=jnp.float32) \
        + b3_ref[...]
    pred = jnp.where(counts > 0.0, pred, 0.0)
    r = lax.broadcasted_iota(jnp.int32, (NUM_GRAPHS, NUM_GRAPHS), 0)
    q = lax.broadcasted_iota(jnp.int32, (NUM_GRAPHS, NUM_GRAPHS), 1)
    eye = (r == q).astype(jnp.float32)
    out_ref[...] = lax.dot_general(
        pred, eye, dimension_numbers=(((0,), (0,)), ((), ())),
        preferred_element_type=jnp.float32)


def kernel(h_v, edge_index, batch, W1, b1, W2, b2, W3, b3):
    del edge_index
    b32 = batch.astype(jnp.int32)
    counts = pl.pallas_call(
        _counts_kernel,
        out_shape=jax.ShapeDtypeStruct((NUM_GRAPHS, 1), jnp.float32),
    )(b32.reshape(1, N_NODES))
    sums = _sc_segment_sum(h_v, b32)
    pred_row = pl.pallas_call(
        _mlp_kernel,
        out_shape=jax.ShapeDtypeStruct((1, NUM_GRAPHS), jnp.float32),
    )(sums, counts, W1, b1.reshape(1, HIDDEN), W2, b2.reshape(1, HIDDEN),
      W3, b3.reshape(1, OUT_DIM))
    return pred_row.reshape(NUM_GRAPHS, OUT_DIM)
